# baseline (speedup 1.0000x reference)
import jax
import jax.numpy as jnp
from jax import lax
from jax.experimental import pallas as pl
from jax.experimental.pallas import tpu as pltpu
from jax.experimental.pallas import tpu_sc as plsc

NC = 2
NS = 16
NW = NC * NS

MAXLEN = 200
EMBED = 64
BATCH = 4096
SEQ = 200

BB = 128
NBT = BATCH // BB
NCHUNK = SEQ * NBT
CPT = NCHUNK // NW


def _body(xt_hbm, tok_hbm, pos_hbm, out_hbm,
          idx_v, rows_a, rows_b, w_a, w_b, pos_v,
          sg_a, sg_b, so_a, so_b, ps):
    wid = lax.axis_index("s") * NC + lax.axis_index("c")
    c0 = wid * CPT

    pltpu.async_copy(pos_hbm, pos_v, ps).wait()
    pltpu.async_copy(xt_hbm.at[pl.ds(c0, CPT)], idx_v, ps).wait()

    ar = jnp.arange(16, dtype=jnp.int32)

    def issue_gather(k, rows, sg):
        pltpu.async_copy(tok_hbm.at[idx_v.at[k]], rows, sg)

    def wait_gather(rows, sg):
        pltpu.make_async_copy(tok_hbm.at[pl.ds(0, BB)], rows, sg).wait()

    def wait_out(w, so):
        pltpu.make_async_copy(out_hbm.at[0, 0, 0], w.at[0], so).wait()

    issue_gather(0, rows_a, sg_a)
    issue_gather(1, rows_b, sg_b)

    def chunk(k, rows, w, sg, so):
        c = c0 + k
        s = c // NBT
        bt = c % NBT
        s_vec = jnp.full((16,), s, jnp.int32)

        wait_gather(rows, sg)

        @pl.when(k >= 2)
        def _():
            for _ in range(8):
                wait_out(w, so)

        for e in range(EMBED):
            e_vec = jnp.full((16,), e, jnp.int32)
            pv = plsc.load_gather(pos_v, [s_vec, e_vec])
            for b0 in range(0, BB, 16):
                col = plsc.load_gather(rows, [ar + b0, e_vec])
                w[e // 8, e % 8, pl.ds(b0, 16)] = col + pv

        for e8 in range(8):
            pltpu.async_copy(w.at[e8], out_hbm.at[s, e8, bt], so)

        @pl.when(k + 2 < CPT)
        def _():
            issue_gather(k + 2, rows, sg)

    def step(m, _):
        chunk(2 * m, rows_a, w_a, sg_a, so_a)
        chunk(2 * m + 1, rows_b, w_b, sg_b, so_b)
        return 0

    lax.fori_loop(0, CPT // 2, step, 0)

    for _ in range(8):
        wait_out(w_a, so_a)
        wait_out(w_b, so_b)


@jax.jit
def _run(xt2, token_table, pos_table):
    mesh = plsc.VectorSubcoreMesh(core_axis_name="c", subcore_axis_name="s")
    f = pl.kernel(
        _body,
        out_type=jax.ShapeDtypeStruct((SEQ, EMBED // 8, NBT, 8, BB), jnp.float32),
        mesh=mesh,
        scratch_types=[
            pltpu.VMEM((CPT, BB), jnp.int32),
            pltpu.VMEM((BB, EMBED), jnp.float32),
            pltpu.VMEM((BB, EMBED), jnp.float32),
            pltpu.VMEM((EMBED // 8, 8, BB), jnp.float32),
            pltpu.VMEM((EMBED // 8, 8, BB), jnp.float32),
            pltpu.VMEM((SEQ, EMBED), jnp.float32),
            pltpu.SemaphoreType.DMA,
            pltpu.SemaphoreType.DMA,
            pltpu.SemaphoreType.DMA,
            pltpu.SemaphoreType.DMA,
            pltpu.SemaphoreType.DMA,
        ],
        compiler_params=pltpu.CompilerParams(
            use_tc_tiling_on_sc=False, needs_layout_passes=False),
    )
    return f(xt2, token_table, pos_table)


def kernel(x, token_table, pos_table):
    xt2 = x.astype(jnp.int32).T.reshape(NCHUNK, BB)
    out5 = _run(xt2, token_table, pos_table)
    return out5.transpose(2, 4, 0, 1, 3).reshape(BATCH, SEQ, EMBED)
